# Initial kernel scaffold; baseline (speedup 1.0000x reference)
import functools

import jax
import jax.numpy as jnp
from jax import lax
from jax.experimental import pallas as pl
from jax.experimental.pallas import tpu as pltpu
from jax.experimental.pallas import tpu_sc as plsc

_NC = 2
_NS = 16
_L = 16
_NW = _NC * _NS

_N, _D = 200000, 64
_TOT = _N * _D
_CHUNK = _TOT // _NW
_SUB = 8000
_NSUB = _CHUNK // _SUB
_STEPS = _SUB // _L

_ROWS = 3125
_LANES = 128


def _base_tc_kernel(mask_ref, base_ref, carry):
    i = pl.program_id(0)

    @pl.when(i == 0)
    def _():
        carry[0] = 0

    base_ref[i] = carry[0]
    carry[0] = carry[0] + jnp.sum(mask_ref[...])


def _compute_bases(mask_i32_3d):
    return pl.pallas_call(
        _base_tc_kernel,
        grid=(_NW,),
        in_specs=[pl.BlockSpec((1, _ROWS, _LANES), lambda i: (i, 0, 0))],
        out_specs=pl.BlockSpec(memory_space=pltpu.SMEM),
        out_shape=jax.ShapeDtypeStruct((_NW,), jnp.int32),
        scratch_shapes=[pltpu.SMEM((1,), jnp.int32)],
        compiler_params=pltpu.CompilerParams(
            dimension_semantics=("arbitrary",),
        ),
    )(mask_i32_3d)


def _sc_body(data_hbm, mask_hbm, imps_hbm, base_hbm, out_hbm,
             dbuf, mbuf, ibuf, obuf, bbuf):
    c = lax.axis_index("c")
    s = lax.axis_index("s")
    wid = s * _NC + c
    start = wid * _CHUNK

    grp = (wid // _L) * _L
    pltpu.sync_copy(base_hbm.at[pl.ds(grp, _L)], bbuf)
    lane = wid - grp
    lanes = lax.iota(jnp.int32, _L)
    off0 = jnp.max(jnp.where(lanes == lane, bbuf[...], 0))

    def outer(b, off):
        blk = start + b * _SUB
        pltpu.sync_copy(data_hbm.at[pl.ds(blk, _SUB)], dbuf)
        pltpu.sync_copy(mask_hbm.at[pl.ds(blk, _SUB)], mbuf)
        off_al = (off // 8) * 8
        rem = off - off_al
        pltpu.sync_copy(imps_hbm.at[pl.ds(off_al, _SUB + 8)], ibuf)

        def inner(i, stv):
            m = mbuf[pl.ds(i * _L, _L)]
            mb = m > 0
            cs = plsc.cumsum(m)
            idx = stv + cs - m
            v = plsc.load_gather(ibuf, [idx], mask=mb)
            d = dbuf[pl.ds(i * _L, _L)]
            obuf[pl.ds(i * _L, _L)] = jnp.where(mb, v, d)
            pc = plsc.all_reduce_population_count(mb)
            return stv + pc

        stv0 = jnp.full((_L,), rem, dtype=jnp.int32)
        stv = lax.fori_loop(0, _STEPS, inner, stv0)
        pltpu.sync_copy(obuf, out_hbm.at[pl.ds(blk, _SUB)])
        return off_al + jnp.max(stv)

    lax.fori_loop(0, _NSUB, outer, off0)


def kernel(data, mask, imps):
    mask_i32 = mask.astype(jnp.int32)
    bases = _compute_bases(mask_i32.reshape(_NW, _ROWS, _LANES))

    nnz = imps.shape[0]
    pad = (-nnz) % 8 + _SUB + 16
    imps_pad = jnp.pad(imps, (0, pad))

    mesh = plsc.VectorSubcoreMesh(
        core_axis_name="c", subcore_axis_name="s",
        num_cores=_NC, num_subcores=_NS,
    )
    sc = functools.partial(
        pl.kernel,
        mesh=mesh,
        out_type=jax.ShapeDtypeStruct((_TOT,), jnp.float32),
        scratch_types=[
            pltpu.VMEM((_SUB,), jnp.float32),
            pltpu.VMEM((_SUB,), jnp.int32),
            pltpu.VMEM((_SUB + 8,), jnp.float32),
            pltpu.VMEM((_SUB,), jnp.float32),
            pltpu.VMEM((_L,), jnp.int32),
        ],
    )(_sc_body)
    out_flat = sc(data.reshape(_TOT), mask_i32.reshape(_TOT), imps_pad, bases)
    return out_flat.reshape(_N, _D)


# speedup vs baseline: 78.8285x; 78.8285x over previous
import functools

import jax
import jax.numpy as jnp
from jax import lax
from jax.experimental import pallas as pl
from jax.experimental.pallas import tpu as pltpu
from jax.experimental.pallas import tpu_sc as plsc

_NC = 2
_NS = 16
_L = 16
_NW = _NC * _NS

_N, _D = 200000, 64
_TOT = _N * _D
_CHUNK = _TOT // _NW
_SUB = 8000
_NSUB = _CHUNK // _SUB
_STEPS = _SUB // _L

_ROWS = 3125
_LANES = 128


def _base_tc_kernel(mask_ref, base_ref, carry):
    i = pl.program_id(0)

    @pl.when(i == 0)
    def _():
        carry[0] = 0

    base_ref[i] = carry[0]
    carry[0] = carry[0] + jnp.sum(mask_ref[...])


def _compute_bases(mask_i32_3d):
    return pl.pallas_call(
        _base_tc_kernel,
        grid=(_NW,),
        in_specs=[pl.BlockSpec((1, _ROWS, _LANES), lambda i: (i, 0, 0))],
        out_specs=pl.BlockSpec(memory_space=pltpu.SMEM),
        out_shape=jax.ShapeDtypeStruct((_NW,), jnp.int32),
        scratch_shapes=[pltpu.SMEM((1,), jnp.int32)],
        compiler_params=pltpu.CompilerParams(
            dimension_semantics=("arbitrary",),
        ),
    )(mask_i32_3d)


def _sc_body(data_hbm, mask_hbm, imps_hbm, base_hbm, out_hbm,
             dbuf, mbuf, ibuf, obuf, bbuf):
    c = lax.axis_index("c")
    s = lax.axis_index("s")
    wid = s * _NC + c
    start = wid * _CHUNK

    grp = (wid // _L) * _L
    pltpu.sync_copy(base_hbm.at[pl.ds(grp, _L)], bbuf)
    lane = jnp.full((_L,), wid - grp, dtype=jnp.int32)
    off0 = plsc.load_gather(bbuf, [lane])[0]

    def outer(b, off):
        blk = start + b * _SUB
        pltpu.sync_copy(data_hbm.at[pl.ds(blk, _SUB)], dbuf)
        pltpu.sync_copy(mask_hbm.at[pl.ds(blk, _SUB)], mbuf)
        off_al = (off // 8) * 8
        rem = off - off_al
        pltpu.sync_copy(imps_hbm.at[pl.ds(off_al, _SUB + 8)], ibuf)

        def inner(i, stv):
            m = mbuf[pl.ds(i * _L, _L)]
            mb = m > 0
            cs = plsc.cumsum(m)
            idx = stv + cs - m
            v = plsc.load_gather(ibuf, [idx], mask=mb)
            d = dbuf[pl.ds(i * _L, _L)]
            obuf[pl.ds(i * _L, _L)] = jnp.where(mb, v, d)
            pc = plsc.all_reduce_population_count(mb)
            return stv + pc

        stv0 = jnp.full((_L,), rem, dtype=jnp.int32)
        stv = lax.fori_loop(0, _STEPS, inner, stv0)
        pltpu.sync_copy(obuf, out_hbm.at[pl.ds(blk, _SUB)])
        return off_al + stv[0]

    lax.fori_loop(0, _NSUB, outer, off0)


def kernel(data, mask, imps):
    mask_i32 = mask.astype(jnp.int32)
    bases = _compute_bases(mask_i32.reshape(_NW, _ROWS, _LANES))

    nnz = imps.shape[0]
    pad = (-nnz) % 8 + _SUB + 16
    imps_pad = jnp.pad(imps, (0, pad))

    mesh = plsc.VectorSubcoreMesh(
        core_axis_name="c", subcore_axis_name="s",
        num_cores=_NC, num_subcores=_NS,
    )
    sc = functools.partial(
        pl.kernel,
        mesh=mesh,
        out_type=jax.ShapeDtypeStruct((_TOT,), jnp.float32),
        scratch_types=[
            pltpu.VMEM((_SUB,), jnp.float32),
            pltpu.VMEM((_SUB,), jnp.int32),
            pltpu.VMEM((_SUB + 8,), jnp.float32),
            pltpu.VMEM((_SUB,), jnp.float32),
            pltpu.VMEM((_L,), jnp.int32),
        ],
        compiler_params=pltpu.CompilerParams(needs_layout_passes=False),
    )(_sc_body)
    out_flat = sc(data.reshape(_TOT), mask_i32.reshape(_TOT), imps_pad, bases)
    return out_flat.reshape(_N, _D)
